# baseline (speedup 1.0000x reference)
import functools

import jax
import jax.numpy as jnp
from jax import lax
from jax.experimental import pallas as pl
from jax.experimental.pallas import tpu as pltpu
from jax.experimental.pallas import tpu_sc as plsc

VOCAB = 1000000
EMBED_DIM = 64
HIDDEN = 256
NUM_CLASSES = 50
BATCH = 4096
SEQ = 200

NC = 2
NS = 16
NW = NC * NS
BPW = BATCH // NW
CHUNK_A = 104
CHUNK_B = SEQ - CHUNK_A
INV_SEQ = 1.0 / SEQ

ROWS_PW = VOCAB // NW
TCH = 248
NFULL = 126
TAIL = 8


def _repack_body(x_hbm, out_hbm, buf_v, flat_v):
    wid = lax.axis_index("s") * NC + lax.axis_index("c")
    base = wid * BPW
    pltpu.sync_copy(x_hbm.at[pl.ds(base, BPW)], buf_v)

    def depad(b, _):
        for i in range(12):
            flat_v[pl.ds(b * SEQ + 16 * i, 16)] = buf_v[b, pl.ds(16 * i, 16)]
        flat_v[pl.ds(b * SEQ + SEQ - 16, 16)] = buf_v[b, pl.ds(SEQ - 16, 16)]
        return 0

    lax.fori_loop(0, BPW, depad, 0)
    pltpu.sync_copy(flat_v, out_hbm.at[pl.ds(base * SEQ, BPW * SEQ)])


@functools.partial(
    pl.kernel,
    out_type=jax.ShapeDtypeStruct((BATCH * SEQ,), jnp.int32),
    mesh=plsc.VectorSubcoreMesh(core_axis_name="c", subcore_axis_name="s"),
    compiler_params=pltpu.CompilerParams(use_tc_tiling_on_sc=True),
    scratch_types=[
        pltpu.VMEM((BPW, SEQ), jnp.int32),
        pltpu.VMEM((BPW * SEQ,), jnp.int32),
    ],
)
def _repack_sc(x_hbm, out_hbm, buf_v, flat_v):
    _repack_body(x_hbm, out_hbm, buf_v, flat_v)


def _tdepad_body(emb_hbm, out_hbm, buf0, buf1, flat0, flat1,
                 ss0, ss1, os0, os1):
    wid = lax.axis_index("s") * NC + lax.axis_index("c")
    a0 = (wid * ROWS_PW) // 8 * 8
    a1 = ((wid + 1) * ROWS_PW) // 8 * 8
    tail = a1 - a0 - NFULL * TCH

    def stage(k, buf, sem):
        pltpu.async_copy(emb_hbm.at[pl.ds(a0 + k * TCH, TCH)], buf, sem)

    def swait(buf, sem):
        pltpu.make_async_copy(emb_hbm.at[pl.ds(0, TCH)], buf, sem).wait()

    def depad(buf, flat):
        def row(r, _):
            for i in range(4):
                flat[pl.ds(r * EMBED_DIM + 16 * i, 16)] = (
                    buf[r, pl.ds(16 * i, 16)])
            return 0
        lax.fori_loop(0, TCH, row, 0, unroll=4)

    def fire_out(k, flat, sem):
        pltpu.async_copy(
            flat, out_hbm.at[pl.ds((a0 + k * TCH) * EMBED_DIM,
                                   TCH * EMBED_DIM)], sem)

    def owait(flat, sem):
        pltpu.make_async_copy(
            flat, out_hbm.at[pl.ds(0, TCH * EMBED_DIM)], sem).wait()

    stage(0, buf0, ss0)
    stage(1, buf1, ss1)

    def pair(kk, _):
        k0 = 2 * kk
        swait(buf0, ss0)
        depad(buf0, flat0)

        @pl.when(k0 + 2 < NFULL)
        def _():
            stage(k0 + 2, buf0, ss0)

        @pl.when(kk > 0)
        def _():
            owait(flat0, os0)
        fire_out(k0, flat0, os0)

        swait(buf1, ss1)
        depad(buf1, flat1)

        @pl.when(k0 + 3 < NFULL)
        def _():
            stage(k0 + 3, buf1, ss1)

        @pl.when(kk > 0)
        def _():
            owait(flat1, os1)
        fire_out(k0 + 1, flat1, os1)
        return 0

    lax.fori_loop(0, NFULL // 2, pair, 0)
    owait(flat0, os0)
    owait(flat1, os1)

    @pl.when(tail > 0)
    def _():
        t0 = a0 + NFULL * TCH
        pltpu.sync_copy(emb_hbm.at[pl.ds(t0, TAIL)],
                        buf0.at[pl.ds(0, TAIL)])

        def row(r, _):
            for i in range(4):
                flat0[pl.ds(r * EMBED_DIM + 16 * i, 16)] = (
                    buf0[r, pl.ds(16 * i, 16)])
            return 0
        lax.fori_loop(0, TAIL, row, 0)
        pltpu.sync_copy(
            flat0.at[pl.ds(0, TAIL * EMBED_DIM)],
            out_hbm.at[pl.ds(t0 * EMBED_DIM, TAIL * EMBED_DIM)])


@functools.partial(
    pl.kernel,
    out_type=jax.ShapeDtypeStruct((VOCAB * EMBED_DIM,), jnp.float32),
    mesh=plsc.VectorSubcoreMesh(core_axis_name="c", subcore_axis_name="s"),
    compiler_params=pltpu.CompilerParams(use_tc_tiling_on_sc=True),
    scratch_types=[
        pltpu.VMEM((TCH, EMBED_DIM), jnp.float32),
        pltpu.VMEM((TCH, EMBED_DIM), jnp.float32),
        pltpu.VMEM((TCH * EMBED_DIM,), jnp.float32),
        pltpu.VMEM((TCH * EMBED_DIM,), jnp.float32),
        pltpu.SemaphoreType.DMA,
        pltpu.SemaphoreType.DMA,
        pltpu.SemaphoreType.DMA,
        pltpu.SemaphoreType.DMA,
    ],
)
def _tdepad_sc(emb_hbm, out_hbm, buf0, buf1, flat0, flat1,
               ss0, ss1, os0, os1):
    _tdepad_body(emb_hbm, out_hbm, buf0, buf1, flat0, flat1,
                 ss0, ss1, os0, os1)


def _pool_body(x_hbm, emb_hbm, out_hbm, idx_v, rows_a, rows_b, pooled_v,
               sem_a, sem_b):
    wid = lax.axis_index("s") * NC + lax.axis_index("c")
    base = wid * BPW
    pltpu.sync_copy(x_hbm.at[pl.ds(base * SEQ, BPW * SEQ)], idx_v)

    def start_a(b):
        pltpu.async_copy(
            emb_hbm.at[idx_v.at[pl.ds(b * SEQ, CHUNK_A)]], rows_a, sem_a)

    def start_b(b):
        pltpu.async_copy(
            emb_hbm.at[idx_v.at[pl.ds(b * SEQ + CHUNK_A, CHUNK_B)]], rows_b,
            sem_b)

    start_a(0)
    start_b(0)

    def accum(rows, init, lo, hi):
        def j_body(j, acc):
            return tuple(
                acc[i] + rows[j, pl.ds(16 * i, 16)] for i in range(4))
        return lax.fori_loop(lo, hi, j_body, init, unroll=8)

    def b_body(b, _):
        pltpu.make_async_copy(
            emb_hbm.at[idx_v.at[pl.ds(0, CHUNK_A)]], rows_a, sem_a).wait()
        acc = tuple(rows_a[0, pl.ds(16 * i, 16)] for i in range(4))
        acc = accum(rows_a, acc, 1, CHUNK_A)

        @pl.when(b < BPW - 1)
        def _():
            start_a(b + 1)

        pltpu.make_async_copy(
            emb_hbm.at[idx_v.at[pl.ds(0, CHUNK_B)]], rows_b, sem_b).wait()
        acc = accum(rows_b, acc, 0, CHUNK_B)

        @pl.when(b < BPW - 1)
        def _():
            start_b(b + 1)

        for i in range(4):
            pooled_v[b, pl.ds(16 * i, 16)] = acc[i] * INV_SEQ
        return 0

    lax.fori_loop(0, BPW, b_body, 0)
    pltpu.sync_copy(pooled_v, out_hbm.at[pl.ds(base, BPW)])


@functools.partial(
    pl.kernel,
    out_type=jax.ShapeDtypeStruct((BATCH, EMBED_DIM), jnp.float32),
    mesh=plsc.VectorSubcoreMesh(core_axis_name="c", subcore_axis_name="s"),
    compiler_params=pltpu.CompilerParams(use_tc_tiling_on_sc=False),
    scratch_types=[
        pltpu.VMEM((BPW * SEQ,), jnp.int32),
        pltpu.VMEM((CHUNK_A, EMBED_DIM), jnp.float32),
        pltpu.VMEM((CHUNK_B, EMBED_DIM), jnp.float32),
        pltpu.VMEM((BPW, EMBED_DIM), jnp.float32),
        pltpu.SemaphoreType.DMA,
        pltpu.SemaphoreType.DMA,
    ],
)
def _pool_sc(x_hbm, emb_hbm, out_hbm, idx_v, rows_a, rows_b, pooled_v,
             sem_a, sem_b):
    _pool_body(x_hbm, emb_hbm, out_hbm, idx_v, rows_a, rows_b, pooled_v,
               sem_a, sem_b)


def _mlp_body(p_ref, w1_ref, b1_ref, w2_ref, b2_ref, o_ref):
    h = jnp.dot(p_ref[...], w1_ref[...], preferred_element_type=jnp.float32)
    h = jnp.maximum(h + b1_ref[...], 0.0)
    o_ref[...] = (
        jnp.dot(h, w2_ref[...], preferred_element_type=jnp.float32)
        + b2_ref[...])


def _mlp_tc(pooled, W1, b1, W2p, b2p):
    return pl.pallas_call(
        _mlp_body,
        out_shape=jax.ShapeDtypeStruct((BATCH, 128), jnp.float32),
    )(pooled, W1, b1, W2p, b2p)


@jax.jit
def kernel(x, emb, W1, b1, W2, b2):
    x_flat = _repack_sc(x.astype(jnp.int32))
    emb_flat = _tdepad_sc(emb)
    pooled = _pool_sc(x_flat, emb_flat.reshape(VOCAB, EMBED_DIM))

    W2p = jnp.pad(W2, ((0, 0), (0, 128 - NUM_CLASSES)))
    b2p = jnp.pad(b2, (0, 128 - NUM_CLASSES)).reshape(1, 128)
    out = _mlp_tc(pooled, W1, b1.reshape(1, HIDDEN), W2p, b2p)
    return out[:, :NUM_CLASSES]
